# Initial kernel scaffold; baseline (speedup 1.0000x reference)
import jax
import jax.numpy as jnp
from jax.experimental import pallas as pl


def kernel(hidden_states, embedding, embedding_bias, output_tokens, presence_penalties, frequency_penalties, temperatures, top_ps, top_ks):
    raise NotImplementedError("write your pallas kernel here")


# speedup vs baseline: 8.8580x; 8.8580x over previous
import functools

import jax
import jax.numpy as jnp
from jax import lax
from jax.experimental import pallas as pl
from jax.experimental.pallas import tpu as pltpu
from jax.experimental.pallas import tpu_sc as plsc

B = 64
D = 1024
V = 100000
L = 128
CH = 1024
NCH = 113
NCH = (V + CH - 1) // CH
PV = NCH * CH
NEG = -1e30
KEY_NEG_INF = -2139095041
KEY_POS_INF = 2139095040
NBIS = 33

_f32 = jnp.float32
_i32 = jnp.int32


def _logits_body(h_ref, e_ref, b_ref, invt_ref, tok_ref, fp_ref, pp_ref,
                 x_ref, m_ref, flat_ref, dval_ref):
    s = pl.program_id(0)
    invt = invt_ref[...]
    logits = lax.dot_general(
        h_ref[...], e_ref[...], (((1,), (1,)), ((), ())),
        preferred_element_type=_f32)
    x = (logits + b_ref[0]) * invt
    col = lax.broadcasted_iota(_i32, (B, CH), 1) + s * CH
    x = jnp.where(col < V, x, NEG)
    x_ref[...] = x

    @pl.when(s == 0)
    def _init():
        m_ref[...] = jnp.full((B, 1), NEG, _f32)

    m_ref[...] = jnp.maximum(m_ref[...], jnp.max(x, axis=1, keepdims=True))

    @pl.when(s == 0)
    def _tokens():
        tok = tok_ref[...]
        eq = tok[:, :, None] == tok[:, None, :]
        cnt = jnp.sum(eq.astype(_f32), axis=2)
        src = lax.broadcasted_iota(_i32, (B, L, L), 2)
        dst = lax.broadcasted_iota(_i32, (B, L, L), 1)
        earlier = jnp.sum((eq & (src < dst)).astype(_f32), axis=2)
        first = earlier == 0.0
        fp = fp_ref[...]
        pp = pp_ref[...]
        dval = jnp.where(first, (fp * cnt + pp) * invt, 0.0)
        row = lax.broadcasted_iota(_i32, (B, L), 0)
        li = lax.broadcasted_iota(_i32, (B, L), 1)
        colt = jnp.where(first, tok, V + li)
        flat_ref[...] = row * PV + colt
        dval_ref[...] = dval


def _run_logits(h, emb, bias3, invt, tok, fp, pp):
    return pl.pallas_call(
        _logits_body,
        grid=(NCH,),
        in_specs=[
            pl.BlockSpec((B, D), lambda s: (0, 0)),
            pl.BlockSpec((CH, D), lambda s: (s, 0)),
            pl.BlockSpec((1, 1, CH), lambda s: (s, 0, 0)),
            pl.BlockSpec((B, 1), lambda s: (0, 0)),
            pl.BlockSpec((B, L), lambda s: (0, 0)),
            pl.BlockSpec((B, 1), lambda s: (0, 0)),
            pl.BlockSpec((B, 1), lambda s: (0, 0)),
        ],
        out_specs=[
            pl.BlockSpec((B, CH), lambda s: (0, s)),
            pl.BlockSpec((B, 1), lambda s: (0, 0)),
            pl.BlockSpec((B, L), lambda s: (0, 0)),
            pl.BlockSpec((B, L), lambda s: (0, 0)),
        ],
        out_shape=[
            jax.ShapeDtypeStruct((B, PV), _f32),
            jax.ShapeDtypeStruct((B, 1), _f32),
            jax.ShapeDtypeStruct((B, L), _i32),
            jax.ShapeDtypeStruct((B, L), _f32),
        ],
        compiler_params=pltpu.CompilerParams(
            dimension_semantics=("arbitrary",)),
    )(h, emb, bias3, invt, tok, fp, pp)


_NUPD = B * L


def _make_sc_penalty():
    info = plsc.get_sparse_core_info()
    nc, ns = info.num_cores, info.num_subcores
    nw = nc * ns
    per_w = _NUPD // nw
    mesh = plsc.VectorSubcoreMesh(core_axis_name="c", subcore_axis_name="s")

    rows_w = B // nw

    @functools.partial(
        pl.kernel,
        mesh=mesh,
        out_type=jax.ShapeDtypeStruct((B * PV,), _f32),
        scratch_types=[
            pltpu.VMEM((per_w,), _i32),
            pltpu.VMEM((per_w,), _f32),
            pltpu.VMEM((per_w,), _f32),
            pltpu.SemaphoreType.DMA,
        ],
    )
    def sc_penalty(x_hbm, idx_hbm, dv_hbm, out_hbm, idx_v, val_v, dv_v, sem):
        wid = lax.axis_index("s") * nc + lax.axis_index("c")
        base = wid * per_w
        rbase = wid * rows_w * PV
        pltpu.sync_copy(x_hbm.at[pl.ds(rbase, rows_w * PV)],
                        out_hbm.at[pl.ds(rbase, rows_w * PV)])
        pltpu.sync_copy(idx_hbm.at[pl.ds(base, per_w)], idx_v)
        pltpu.sync_copy(dv_hbm.at[pl.ds(base, per_w)], dv_v)
        pltpu.async_copy(out_hbm.at[idx_v], val_v, sem).wait()
        for j in range(per_w // 16):
            sl = pl.ds(j * 16, 16)
            val_v[sl] = val_v[sl] - dv_v[sl]
        pltpu.async_copy(val_v, out_hbm.at[idx_v], sem).wait()

    return sc_penalty


def _decode_key(k):
    bits = jnp.where(k >= 0, k, k ^ jnp.int32(0x7FFFFFFF))
    return lax.bitcast_convert_type(bits, _f32)


def _avg(lo, hi):
    return (lo >> 1) + (hi >> 1) + (lo & hi & 1)


_W = 128


def _lanesum(v):
    acc = v[:, 0:_W]
    for i in range(1, CH // _W):
        acc = acc + v[:, i * _W:(i + 1) * _W]
    return acc


def _rowsum(a):
    return jnp.sum(a, axis=1, keepdims=True)


def _c1_body(x_ref, m_ref, k_ref, p_ref, cval_ref, z_ref, e_scr):
    m = m_ref[...]
    zacc = jnp.zeros((B, _W), _f32)

    def p1(c, acc):
        sl = pl.ds(pl.multiple_of(c * CH, CH), CH)
        ec = jnp.exp(x_ref[:, sl] - m)
        e_scr[:, sl] = ec
        return acc + _lanesum(ec)

    S = _rowsum(lax.fori_loop(0, NCH, p1, zacc))
    pS = p_ref[...] * S
    k = k_ref[...]

    def bis(_, carry):
        lo1, hi1, lo2, hi2 = carry
        mid1 = _avg(lo1, hi1)
        mid2 = _avg(lo2, hi2)
        t1 = _decode_key(mid1)
        t2 = _decode_key(mid2)

        def scan(c, acc):
            a1, a2 = acc
            sl = pl.ds(pl.multiple_of(c * CH, CH), CH)
            xc = x_ref[:, sl]
            ec = e_scr[:, sl]
            a1 = a1 + _lanesum((xc >= t1).astype(_f32))
            a2 = a2 + _lanesum(jnp.where(xc > t2, ec, 0.0))
            return (a1, a2)

        a1, a2 = lax.fori_loop(0, NCH, scan, (zacc, zacc))
        cnt = _rowsum(a1)
        egt = _rowsum(a2)
        ok1 = cnt >= k
        lo1 = jnp.where(ok1, mid1, lo1)
        hi1 = jnp.where(ok1, hi1, mid1)
        ok2 = egt <= pS
        hi2 = jnp.where(ok2, mid2, hi2)
        lo2 = jnp.where(ok2, lo2, mid2)
        return lo1, hi1, lo2, hi2

    klo = jnp.full((B, 1), KEY_NEG_INF, _i32)
    khi = jnp.full((B, 1), KEY_POS_INF, _i32)
    lo1, hi1, lo2, hi2 = lax.fori_loop(0, NBIS, bis, (klo, khi, klo, khi))
    c_val = jnp.maximum(_decode_key(lo1), _decode_key(hi2))

    def zscan(c, acc):
        sl = pl.ds(pl.multiple_of(c * CH, CH), CH)
        xc = x_ref[:, sl]
        ec = e_scr[:, sl]
        return acc + _lanesum(jnp.where(xc >= c_val, ec, 0.0))

    Z = _rowsum(lax.fori_loop(0, NCH, zscan, zacc))
    cval_ref[...] = c_val
    z_ref[...] = Z


def _run_c1(x2, m, kf, p):
    return pl.pallas_call(
        _c1_body,
        grid=(1,),
        in_specs=[
            pl.BlockSpec((B, PV), lambda s: (0, 0)),
            pl.BlockSpec((B, 1), lambda s: (0, 0)),
            pl.BlockSpec((B, 1), lambda s: (0, 0)),
            pl.BlockSpec((B, 1), lambda s: (0, 0)),
        ],
        out_specs=[
            pl.BlockSpec((B, 1), lambda s: (0, 0)),
            pl.BlockSpec((B, 1), lambda s: (0, 0)),
        ],
        out_shape=[
            jax.ShapeDtypeStruct((B, 1), _f32),
            jax.ShapeDtypeStruct((B, 1), _f32),
        ],
        scratch_shapes=[pltpu.VMEM((B, PV), _f32)],
        compiler_params=pltpu.CompilerParams(
            dimension_semantics=("arbitrary",)),
    )(x2, m, kf, p)


def _c2_body(x_ref, m_ref, cval_ref, z_ref, o_ref):
    x = x_ref[...]
    e = jnp.exp(x - m_ref[...])
    o_ref[...] = jnp.where(x >= cval_ref[...], e / z_ref[...], 0.0)


def _run_c2(x2, m, cval, z):
    return pl.pallas_call(
        _c2_body,
        grid=(NCH,),
        in_specs=[
            pl.BlockSpec((B, CH), lambda s: (0, s)),
            pl.BlockSpec((B, 1), lambda s: (0, 0)),
            pl.BlockSpec((B, 1), lambda s: (0, 0)),
            pl.BlockSpec((B, 1), lambda s: (0, 0)),
        ],
        out_specs=pl.BlockSpec((B, CH), lambda s: (0, s)),
        out_shape=jax.ShapeDtypeStruct((B, V), _f32),
        compiler_params=pltpu.CompilerParams(
            dimension_semantics=("arbitrary",)),
    )(x2, m, cval, z)


def kernel(hidden_states, embedding, embedding_bias, output_tokens,
           presence_penalties, frequency_penalties, temperatures,
           top_ps, top_ks):
    invt = (1.0 / temperatures).astype(_f32).reshape(B, 1)
    fp = frequency_penalties.astype(_f32).reshape(B, 1)
    pp = presence_penalties.astype(_f32).reshape(B, 1)
    kf = top_ks.astype(_f32).reshape(B, 1)
    p = top_ps.astype(_f32).reshape(B, 1)
    bias3 = jnp.concatenate(
        [embedding_bias.astype(_f32), jnp.zeros((PV - V,), _f32)]
    ).reshape(NCH, 1, CH)
    tok = output_tokens.astype(_i32)

    x, m, flat, dval = _run_logits(
        hidden_states.astype(_f32), embedding.astype(_f32), bias3, invt,
        tok, fp, pp)

    xp = _make_sc_penalty()(x.reshape(-1), flat.reshape(-1), dval.reshape(-1))
    x2 = xp.reshape(B, PV)

    cval, z = _run_c1(x2, m, kf, p)
    return _run_c2(x2, m, cval, z)
